# Initial kernel scaffold; baseline (speedup 1.0000x reference)
import jax
import jax.numpy as jnp
from jax.experimental import pallas as pl


def kernel(x, W_ih, W_hh, b_ih, b_hh, fc_w, fc_b):
    raise NotImplementedError("write your pallas kernel here")


# speedup vs baseline: 2.8423x; 2.8423x over previous
import jax
import jax.numpy as jnp
from jax.experimental import pallas as pl
from jax.experimental.pallas import tpu as pltpu

_BB = 128


def _rnn_kernel(xt_ref, wih_ref, wcomb_ref, bias_ref, fcb_ref, out_ref,
                xp_ref, h_ref):
    C = xt_ref.shape[1]
    H = wih_ref.shape[2]
    s = pl.program_id(1)

    @pl.when(s == 0)
    def _():
        h_ref[...] = jnp.zeros_like(h_ref)

    x2d = xt_ref[0].reshape(C * _BB, xt_ref.shape[3])
    xp_ref[...] = (
        jnp.dot(x2d, wih_ref[0], preferred_element_type=jnp.float32)
        + bias_ref[0]
    )

    wc = wcomb_ref[0]
    fcb = fcb_ref[0]
    h = h_ref[...]

    for t in range(C):
        z = jnp.dot(h, wc, preferred_element_type=jnp.float32)
        if t > 0:
            y = z[:, H:H + 1] + fcb
            out_ref[0, :, t - 1:t] = 1.0 / (1.0 + jnp.exp(-y))
        h = jnp.tanh(xp_ref[pl.ds(t * _BB, _BB), :] + z[:, :H])

    h_ref[...] = h
    zf = jnp.dot(h, wc, preferred_element_type=jnp.float32)
    yf = zf[:, H:H + 1] + fcb
    out_ref[0, :, C - 1:C] = 1.0 / (1.0 + jnp.exp(-yf))


def kernel(x, W_ih, W_hh, b_ih, b_hh, fc_w, fc_b):
    B, T, D = x.shape
    S, H = W_ih.shape[0], W_ih.shape[1]
    C = T // S
    NB = B // _BB

    xt = x.reshape(B, S, C, D).transpose(1, 2, 0, 3)
    wihT = jnp.swapaxes(W_ih, 1, 2)
    whhT = jnp.swapaxes(W_hh, 1, 2)
    pad = jnp.zeros((S, H, 128 - H - 1), x.dtype)
    wcomb = jnp.concatenate([whhT, fc_w[:, :, None], pad], axis=2)
    bias = (b_ih + b_hh)[:, None, :]
    fcb = fc_b[:, None, None]

    out = pl.pallas_call(
        _rnn_kernel,
        out_shape=jax.ShapeDtypeStruct((S, B, C), x.dtype),
        grid=(NB, S),
        in_specs=[
            pl.BlockSpec((1, C, _BB, D), lambda b, s: (s, 0, b, 0)),
            pl.BlockSpec((1, D, H), lambda b, s: (s, 0, 0)),
            pl.BlockSpec((1, H, 128), lambda b, s: (s, 0, 0)),
            pl.BlockSpec((1, 1, H), lambda b, s: (s, 0, 0)),
            pl.BlockSpec((1, 1, 1), lambda b, s: (s, 0, 0)),
        ],
        out_specs=pl.BlockSpec((1, _BB, C), lambda b, s: (s, b, 0)),
        scratch_shapes=[
            pltpu.VMEM((C * _BB, H), jnp.float32),
            pltpu.VMEM((_BB, H), jnp.float32),
        ],
        compiler_params=pltpu.CompilerParams(
            dimension_semantics=("parallel", "arbitrary"),
            vmem_limit_bytes=48 * 1024 * 1024,
        ),
        name="tv_rnn_fused",
    )(xt, wihT, wcomb, bias, fcb)

    return out.transpose(1, 0, 2).reshape(B, T, 1)


# speedup vs baseline: 4.5850x; 1.6132x over previous
import jax
import jax.numpy as jnp
from jax.experimental import pallas as pl
from jax.experimental.pallas import tpu as pltpu

_BB = 128


def _issue_seg_dmas(x_hbm, xbuf, sem, b0, seg, slot, C):
    for t in range(C):
        pltpu.make_async_copy(
            x_hbm.at[pl.ds(b0, _BB), seg * C + t, :],
            xbuf.at[slot, t],
            sem.at[slot],
        ).start()


def _rnn_kernel(x_hbm, wih_ref, wcomb_ref, bias_ref, fcb_ref, out_ref,
                xbuf, xp_ref, h_ref, sem):
    C = xbuf.shape[1]
    D = xbuf.shape[3]
    H = wih_ref.shape[2]
    S = pl.num_programs(1)
    b = pl.program_id(0)
    s = pl.program_id(1)
    b0 = b * _BB
    slot = jax.lax.rem(s, 2)

    @pl.when(s == 0)
    def _():
        h_ref[...] = jnp.zeros_like(h_ref)
        _issue_seg_dmas(x_hbm, xbuf, sem, b0, s, slot, C)

    @pl.when(s < S - 1)
    def _():
        _issue_seg_dmas(x_hbm, xbuf, sem, b0, s + 1, 1 - slot, C)

    for t in range(C):
        pltpu.make_async_copy(
            x_hbm.at[pl.ds(b0, _BB), s * C + t, :],
            xbuf.at[slot, t],
            sem.at[slot],
        ).wait()

    x2d = xbuf[slot].reshape(C * _BB, D)
    xp_ref[...] = (
        jnp.dot(x2d, wih_ref[0], preferred_element_type=jnp.float32)
        + bias_ref[0]
    )

    wc = wcomb_ref[0]
    fcb = fcb_ref[0]
    h = h_ref[...]

    for t in range(C):
        z = jnp.dot(h, wc, preferred_element_type=jnp.float32)
        if t > 0:
            y = z[:, H:H + 1] + fcb
            out_ref[0, :, t - 1:t] = 1.0 / (1.0 + jnp.exp(-y))
        h = jnp.tanh(xp_ref[pl.ds(t * _BB, _BB), :] + z[:, :H])

    h_ref[...] = h
    zf = jnp.dot(h, wc, preferred_element_type=jnp.float32)
    yf = zf[:, H:H + 1] + fcb
    out_ref[0, :, C - 1:C] = 1.0 / (1.0 + jnp.exp(-yf))


def kernel(x, W_ih, W_hh, b_ih, b_hh, fc_w, fc_b):
    B, T, D = x.shape
    S, H = W_ih.shape[0], W_ih.shape[1]
    C = T // S
    NB = B // _BB

    wihT = jnp.swapaxes(W_ih, 1, 2)
    whhT = jnp.swapaxes(W_hh, 1, 2)
    pad = jnp.zeros((S, H, 128 - H - 1), x.dtype)
    wcomb = jnp.concatenate([whhT, fc_w[:, :, None], pad], axis=2)
    bias = (b_ih + b_hh)[:, None, :]
    fcb = fc_b[:, None, None]

    out = pl.pallas_call(
        _rnn_kernel,
        out_shape=jax.ShapeDtypeStruct((S, B, C), x.dtype),
        grid=(NB, S),
        in_specs=[
            pl.BlockSpec(memory_space=pl.ANY),
            pl.BlockSpec((1, D, H), lambda b, s: (s, 0, 0)),
            pl.BlockSpec((1, H, 128), lambda b, s: (s, 0, 0)),
            pl.BlockSpec((1, 1, H), lambda b, s: (s, 0, 0)),
            pl.BlockSpec((1, 1, 1), lambda b, s: (s, 0, 0)),
        ],
        out_specs=pl.BlockSpec((1, _BB, C), lambda b, s: (s, b, 0)),
        scratch_shapes=[
            pltpu.VMEM((2, C, _BB, D), jnp.float32),
            pltpu.VMEM((C * _BB, H), jnp.float32),
            pltpu.VMEM((_BB, H), jnp.float32),
            pltpu.SemaphoreType.DMA((2,)),
        ],
        compiler_params=pltpu.CompilerParams(
            dimension_semantics=("parallel", "arbitrary"),
            vmem_limit_bytes=48 * 1024 * 1024,
        ),
        name="tv_rnn_fused",
    )(x, wihT, wcomb, bias, fcb)

    return out.transpose(1, 0, 2).reshape(B, T, 1)
